# baseline (speedup 1.0000x reference)
import jax
import jax.numpy as jnp
from jax import lax
from jax.experimental import pallas as pl
from jax.experimental.pallas import tpu as pltpu

TEM_NUM = 3
BPS = 16


def _fused_kernel(pts_ref, tid_ref, w1t_ref, w2t_ref, out_ref):
    NB = pts_ref.shape[1]
    N = NB // BPS
    ptsT = pts_ref[...].astype(jnp.bfloat16)
    at = tid_ref[...]
    w1t = w1t_ref[...]
    w2t = w2t_ref[...]

    zero = jnp.bfloat16(0)
    h = jnp.maximum(
        jnp.dot(w1t, ptsT, preferred_element_type=jnp.float32).astype(jnp.bfloat16),
        zero,
    )
    h = jnp.maximum(
        jnp.dot(w2t, h, preferred_element_type=jnp.float32).astype(jnp.bfloat16),
        zero,
    )

    n_t = TEM_NUM - 1
    tvec = lax.broadcasted_iota(jnp.int32, (n_t, N), 0) + 1
    for i in range(BPS):
        at_i = at[:, i * N : (i + 1) * N]
        masks = (jnp.broadcast_to(at_i, (n_t, N)) == tvec).astype(jnp.bfloat16)
        sums = lax.dot_general(
            h[:, i * N : (i + 1) * N], masks, (((1,), (1,)), ((), ())),
            preferred_element_type=jnp.float32,
        )
        for t in range(1, TEM_NUM):
            c = jnp.maximum(jnp.sum((at_i == t).astype(jnp.float32)), 1.0)
            out_ref[i, :, t - 1 : t] = sums[:, t - 1 : t] / c


def kernel(points, time_ids, W1, W2):
    B, N, D = points.shape
    H = W1.shape[1]
    n_t = TEM_NUM - 1
    ptsT = points.transpose(2, 0, 1).reshape(D, B * N)
    tids2 = time_ids.reshape(1, B * N)

    out = pl.pallas_call(
        _fused_kernel,
        grid=(B // BPS,),
        in_specs=[
            pl.BlockSpec((D, BPS * N), lambda g: (0, g)),
            pl.BlockSpec((1, BPS * N), lambda g: (0, g)),
            pl.BlockSpec((H, D), lambda g: (0, 0)),
            pl.BlockSpec((H, H), lambda g: (0, 0)),
        ],
        out_specs=pl.BlockSpec((BPS, H, n_t), lambda g: (g, 0, 0)),
        out_shape=jax.ShapeDtypeStruct((B, H, n_t), jnp.float32),
        compiler_params=pltpu.CompilerParams(vmem_limit_bytes=128 * 1024 * 1024),
    )(ptsT, tids2, W1.T.astype(jnp.bfloat16), W2.T.astype(jnp.bfloat16))

    return out.transpose(2, 0, 1)


# speedup vs baseline: 1.0300x; 1.0300x over previous
import jax
import jax.numpy as jnp
from jax import lax
from jax.experimental import pallas as pl

TEM_NUM = 3
BPS = 8


def _fused_kernel(pts_ref, tid_ref, w1t_ref, w2t_ref, out_ref):
    NB = pts_ref.shape[1]
    N = NB // BPS
    n_t = TEM_NUM - 1
    ptsT = pts_ref[...].astype(jnp.bfloat16)
    at = tid_ref[...]
    w1t = w1t_ref[...].astype(jnp.bfloat16)
    w2t = w2t_ref[...].astype(jnp.bfloat16)

    zero = jnp.bfloat16(0)
    h1 = jnp.maximum(
        jnp.dot(w1t, ptsT, preferred_element_type=jnp.float32).astype(jnp.bfloat16),
        zero,
    )

    for i in range(BPS):
        h2 = jnp.maximum(
            jnp.dot(
                w2t, h1[:, i * N : (i + 1) * N],
                preferred_element_type=jnp.float32,
            ),
            0.0,
        )
        at_i = at[:, i * N : (i + 1) * N]
        for t in range(1, TEM_NUM):
            m = at_i == t
            s = jnp.sum(
                jnp.where(jnp.broadcast_to(m, h2.shape), h2, 0.0),
                axis=1,
                keepdims=True,
            )
            c = jnp.maximum(jnp.sum(m.astype(jnp.float32)), 1.0)
            out_ref[i, :, t - 1 : t] = s / c


def kernel(points, time_ids, W1, W2):
    B, N, D = points.shape
    H = W1.shape[1]
    n_t = TEM_NUM - 1
    ptsT = points.transpose(2, 0, 1).reshape(D, B * N)
    tids2 = time_ids.reshape(1, B * N)

    out = pl.pallas_call(
        _fused_kernel,
        grid=(B // BPS,),
        in_specs=[
            pl.BlockSpec((D, BPS * N), lambda g: (0, g)),
            pl.BlockSpec((1, BPS * N), lambda g: (0, g)),
            pl.BlockSpec((H, D), lambda g: (0, 0)),
            pl.BlockSpec((H, H), lambda g: (0, 0)),
        ],
        out_specs=pl.BlockSpec((BPS, H, n_t), lambda g: (g, 0, 0)),
        out_shape=jax.ShapeDtypeStruct((B, H, n_t), jnp.float32),
    )(ptsT, tids2, W1.T, W2.T)

    return out.transpose(2, 0, 1)


# speedup vs baseline: 1.0462x; 1.0157x over previous
import jax
import jax.numpy as jnp
from jax import lax
from jax.experimental import pallas as pl

TEM_NUM = 3
BPS = 4


def _fused_kernel(pts_ref, tid_ref, w1t_ref, w2t_ref, out_ref):
    NB = pts_ref.shape[1]
    N = NB // BPS
    ptsT = pts_ref[...].astype(jnp.bfloat16)
    at = tid_ref[...]
    w1t = w1t_ref[...].astype(jnp.bfloat16)
    w2t = w2t_ref[...].astype(jnp.bfloat16)

    zero = jnp.bfloat16(0)
    h = jnp.maximum(
        jnp.dot(w1t, ptsT, preferred_element_type=jnp.float32).astype(jnp.bfloat16),
        zero,
    )
    h = jnp.maximum(
        jnp.dot(w2t, h, preferred_element_type=jnp.float32).astype(jnp.bfloat16),
        zero,
    )

    n_t = TEM_NUM - 1
    tvec = lax.broadcasted_iota(jnp.int32, (n_t, N), 0) + 1
    for i in range(BPS):
        at_i = at[:, i * N : (i + 1) * N]
        masks = (jnp.broadcast_to(at_i, (n_t, N)) == tvec).astype(jnp.bfloat16)
        sums = lax.dot_general(
            h[:, i * N : (i + 1) * N], masks, (((1,), (1,)), ((), ())),
            preferred_element_type=jnp.float32,
        )
        for t in range(1, TEM_NUM):
            c = jnp.maximum(jnp.sum((at_i == t).astype(jnp.float32)), 1.0)
            out_ref[i, :, t - 1 : t] = sums[:, t - 1 : t] / c


def kernel(points, time_ids, W1, W2):
    B, N, D = points.shape
    H = W1.shape[1]
    n_t = TEM_NUM - 1
    ptsT = points.transpose(2, 0, 1).reshape(D, B * N)
    tids2 = time_ids.reshape(1, B * N)

    out = pl.pallas_call(
        _fused_kernel,
        grid=(B // BPS,),
        in_specs=[
            pl.BlockSpec((D, BPS * N), lambda g: (0, g)),
            pl.BlockSpec((1, BPS * N), lambda g: (0, g)),
            pl.BlockSpec((H, D), lambda g: (0, 0)),
            pl.BlockSpec((H, H), lambda g: (0, 0)),
        ],
        out_specs=pl.BlockSpec((BPS, H, n_t), lambda g: (g, 0, 0)),
        out_shape=jax.ShapeDtypeStruct((B, H, n_t), jnp.float32),
    )(ptsT, tids2, W1.T, W2.T)

    return out.transpose(2, 0, 1)


# speedup vs baseline: 1.0475x; 1.0013x over previous
import jax
import jax.numpy as jnp
from jax import lax
from jax.experimental import pallas as pl

TEM_NUM = 3
BPS = 8


def _fused_kernel(pts_ref, tid_ref, w1t_ref, w2t_ref, out_ref):
    NB = pts_ref.shape[1]
    N = NB // BPS
    ptsT = pts_ref[...].astype(jnp.bfloat16)
    at = tid_ref[...]
    w1t = w1t_ref[...].astype(jnp.bfloat16)
    w2t = w2t_ref[...].astype(jnp.bfloat16)

    zero = jnp.bfloat16(0)
    h = jnp.maximum(
        jnp.dot(w1t, ptsT, preferred_element_type=jnp.float32).astype(jnp.bfloat16),
        zero,
    )
    h = jnp.maximum(
        jnp.dot(w2t, h, preferred_element_type=jnp.float32).astype(jnp.bfloat16),
        zero,
    )

    n_t = TEM_NUM - 1
    tvec = lax.broadcasted_iota(jnp.int32, (n_t, N), 0) + 1
    for i in range(BPS):
        at_i = at[:, i * N : (i + 1) * N]
        masks = (jnp.broadcast_to(at_i, (n_t, N)) == tvec).astype(jnp.bfloat16)
        sums = lax.dot_general(
            h[:, i * N : (i + 1) * N], masks, (((1,), (1,)), ((), ())),
            preferred_element_type=jnp.float32,
        )
        for t in range(1, TEM_NUM):
            c = jnp.maximum(jnp.sum((at_i == t).astype(jnp.float32)), 1.0)
            out_ref[i, :, t - 1 : t] = sums[:, t - 1 : t] / c


def kernel(points, time_ids, W1, W2):
    B, N, D = points.shape
    H = W1.shape[1]
    n_t = TEM_NUM - 1
    ptsT = points.transpose(2, 0, 1).reshape(D, B * N)
    tids2 = time_ids.reshape(1, B * N)

    out = pl.pallas_call(
        _fused_kernel,
        grid=(B // BPS,),
        in_specs=[
            pl.BlockSpec((D, BPS * N), lambda g: (0, g)),
            pl.BlockSpec((1, BPS * N), lambda g: (0, g)),
            pl.BlockSpec((H, D), lambda g: (0, 0)),
            pl.BlockSpec((H, H), lambda g: (0, 0)),
        ],
        out_specs=pl.BlockSpec((BPS, H, n_t), lambda g: (g, 0, 0)),
        out_shape=jax.ShapeDtypeStruct((B, H, n_t), jnp.float32),
    )(ptsT, tids2, W1.T, W2.T)

    return out.transpose(2, 0, 1)
